# Initial kernel scaffold; baseline (speedup 1.0000x reference)
import jax
import jax.numpy as jnp
from jax.experimental import pallas as pl


def kernel(x, p):
    raise NotImplementedError("write your pallas kernel here")


# speedup vs baseline: 2.8459x; 2.8459x over previous
import functools

import jax
import jax.numpy as jnp
from jax import lax
from jax.experimental import pallas as pl
from jax.experimental.pallas import tpu as pltpu
from jax.experimental.pallas import tpu_sc as plsc

BATCH = 4
N = 4096
D = 2048
KTOP = 1024

RB = 512


def _scores_body(x_ref, p_ref, s_ref):
    pv = p_ref[...]
    nrm = jnp.sqrt(jnp.sum(pv * pv))
    s = jnp.dot(x_ref[0], pv, preferred_element_type=jnp.float32)
    s_ref[0] = s / nrm


def _scores(x, p):
    return pl.pallas_call(
        _scores_body,
        grid=(BATCH, N // RB),
        in_specs=[
            pl.BlockSpec((1, RB, D), lambda b, r: (b, r, 0)),
            pl.BlockSpec((D, 1), lambda b, r: (0, 0)),
        ],
        out_specs=pl.BlockSpec((1, RB, 1), lambda b, r: (b, r, 0)),
        out_shape=jax.ShapeDtypeStruct((BATCH, N, 1), jnp.float32),
    )(x, p)


CCH = 256
KCH = 256


def _topk_body(scol_ref, srow_ref, idx_ref, scale_ref):
    srow = srow_ref[0]

    def rank_chunk(c, acc):
        sj = scol_ref[0, pl.ds(c * CCH, CCH), :]
        jj = lax.broadcasted_iota(jnp.int32, (CCH, N), 0) + c * CCH
        ii = lax.broadcasted_iota(jnp.int32, (CCH, N), 1)
        cmp = (sj > srow) | ((sj == srow) & (jj < ii))
        return acc + jnp.sum(cmp.astype(jnp.int32), axis=0, keepdims=True)

    rank = lax.fori_loop(0, N // CCH, rank_chunk,
                         jnp.zeros((1, N), jnp.int32))

    def sel_chunk(kc, _):
        kk = lax.broadcasted_iota(jnp.int32, (KCH, N), 0) + kc * KCH
        ii = lax.broadcasted_iota(jnp.int32, (KCH, N), 1)
        eq = rank == kk
        idxc = jnp.sum(jnp.where(eq, ii, 0), axis=1, keepdims=True)
        valc = jnp.sum(jnp.where(eq, srow, 0.0), axis=1, keepdims=True)
        idx_ref[0, pl.ds(kc * KCH, KCH), :] = idxc
        scale_ref[0, pl.ds(kc * KCH, KCH), :] = jnp.broadcast_to(
            jnp.tanh(valc), (KCH, SC_L))
        return 0

    lax.fori_loop(0, KTOP // KCH, sel_chunk, 0)


def _topk(scol, srow):
    return pl.pallas_call(
        _topk_body,
        grid=(BATCH,),
        in_specs=[
            pl.BlockSpec((1, N, 1), lambda b: (b, 0, 0)),
            pl.BlockSpec((1, 1, N), lambda b: (b, 0, 0)),
        ],
        out_specs=[
            pl.BlockSpec((1, KTOP, 1), lambda b: (b, 0, 0)),
            pl.BlockSpec((1, KTOP, SC_L), lambda b: (b, 0, 0)),
        ],
        out_shape=[
            jax.ShapeDtypeStruct((BATCH, KTOP, 1), jnp.int32),
            jax.ShapeDtypeStruct((BATCH, KTOP, SC_L), jnp.float32),
        ],
    )(scol, srow)


SC_NC = 2
SC_NS = 16
SC_L = 16
NW = SC_NC * SC_NS
RPW = (BATCH * KTOP) // NW
CH = 16
NCH = RPW // CH


def _gather_body(xf_hbm, idx_hbm, scale_hbm, out_hbm,
                 idx_v, scale_v, gidx_v, buf, sem):
    wid = lax.axis_index("s") * SC_NC + lax.axis_index("c")
    base = wid * RPW
    boff = (base // KTOP) * N
    pltpu.sync_copy(idx_hbm.at[wid], idx_v)
    pltpu.sync_copy(scale_hbm.at[wid], scale_v)
    for c in range(NCH):
        gidx_v[...] = idx_v[c, :] + boff
        pltpu.async_copy(xf_hbm.at[gidx_v], buf, sem).wait()
        for r in range(CH):
            sv = scale_v[c * CH + r, :]

            def colbody(j, _):
                sl = pl.ds(j * SC_L, SC_L)
                buf[r, sl] = buf[r, sl] * sv
                return 0

            lax.fori_loop(0, D // SC_L, colbody, 0, unroll=8)
        pltpu.sync_copy(buf, out_hbm.at[pl.ds(base + c * CH, CH)])


@functools.cache
def _make_gather_scale():
    return functools.partial(
        pl.kernel,
        mesh=plsc.VectorSubcoreMesh(core_axis_name="c", subcore_axis_name="s"),
        out_type=jax.ShapeDtypeStruct((BATCH * KTOP, D), jnp.float32),
        scratch_types=[
            pltpu.VMEM((NCH, SC_L), jnp.int32),
            pltpu.VMEM((RPW, SC_L), jnp.float32),
            pltpu.VMEM((SC_L,), jnp.int32),
            pltpu.VMEM((CH, D), jnp.float32),
            pltpu.SemaphoreType.DMA,
        ],
    )(_gather_body)


def kernel(x, p):
    scol = _scores(x, p)
    srow = scol.reshape(BATCH, 1, N)
    idxc, scalec = _topk(scol, srow)
    xf = x.reshape(BATCH * N, D)
    idx3 = idxc.reshape(NW, NCH, SC_L)
    scale3 = scalec.reshape(NW, RPW, SC_L)
    outf = _make_gather_scale()(xf, idx3, scale3)
    return outf.reshape(BATCH, KTOP, D)


# speedup vs baseline: 2.8702x; 1.0085x over previous
import functools

import jax
import jax.numpy as jnp
from jax import lax
from jax.experimental import pallas as pl
from jax.experimental.pallas import tpu as pltpu
from jax.experimental.pallas import tpu_sc as plsc

BATCH = 4
N = 4096
D = 2048
KTOP = 1024

RB = 512


def _scores_body(x_ref, pt_ref, s_ref):
    pv = pt_ref[...]
    nrm = jnp.sqrt(jnp.sum(pv * pv))
    st = lax.dot_general(pv, x_ref[0], (((1,), (1,)), ((), ())),
                         preferred_element_type=jnp.float32)
    s_ref[0] = st.T / nrm


def _scores(x, pt):
    return pl.pallas_call(
        _scores_body,
        grid=(BATCH, N // RB),
        in_specs=[
            pl.BlockSpec((1, RB, D), lambda b, r: (b, r, 0)),
            pl.BlockSpec((1, D), lambda b, r: (0, 0)),
        ],
        out_specs=pl.BlockSpec((1, RB, 1), lambda b, r: (b, r, 0)),
        out_shape=jax.ShapeDtypeStruct((BATCH, N, 1), jnp.float32),
    )(x, pt)


CCH = 256
KCH = 256


def _topk_body(scol_ref, srow_ref, idx_ref, scale_ref):
    srow = srow_ref[0]

    def rank_chunk(c, acc):
        sj = scol_ref[0, pl.ds(c * CCH, CCH), :]
        jj = lax.broadcasted_iota(jnp.int32, (CCH, N), 0) + c * CCH
        ii = lax.broadcasted_iota(jnp.int32, (CCH, N), 1)
        cmp = (sj > srow) | ((sj == srow) & (jj < ii))
        return acc + jnp.sum(cmp.astype(jnp.int32), axis=0, keepdims=True)

    rank = lax.fori_loop(0, N // CCH, rank_chunk,
                         jnp.zeros((1, N), jnp.int32))

    def sel_chunk(kc, _):
        kk = lax.broadcasted_iota(jnp.int32, (KCH, N), 0) + kc * KCH
        ii = lax.broadcasted_iota(jnp.int32, (KCH, N), 1)
        eq = rank == kk
        idxc = jnp.sum(jnp.where(eq, ii, 0), axis=1, keepdims=True)
        valc = jnp.sum(jnp.where(eq, srow, 0.0), axis=1, keepdims=True)
        idx_ref[0, pl.ds(kc * KCH, KCH), :] = idxc
        scale_ref[0, pl.ds(kc * KCH, KCH), :] = jnp.broadcast_to(
            jnp.tanh(valc), (KCH, SC_L))
        return 0

    lax.fori_loop(0, KTOP // KCH, sel_chunk, 0)


def _topk(scol, srow):
    return pl.pallas_call(
        _topk_body,
        grid=(BATCH,),
        in_specs=[
            pl.BlockSpec((1, N, 1), lambda b: (b, 0, 0)),
            pl.BlockSpec((1, 1, N), lambda b: (b, 0, 0)),
        ],
        out_specs=[
            pl.BlockSpec((1, KTOP, 1), lambda b: (b, 0, 0)),
            pl.BlockSpec((1, KTOP, SC_L), lambda b: (b, 0, 0)),
        ],
        out_shape=[
            jax.ShapeDtypeStruct((BATCH, KTOP, 1), jnp.int32),
            jax.ShapeDtypeStruct((BATCH, KTOP, SC_L), jnp.float32),
        ],
    )(scol, srow)


SC_NC = 2
SC_NS = 16
SC_L = 16
NW = SC_NC * SC_NS
RPW = (BATCH * KTOP) // NW
CH = 16
NCH = RPW // CH


def _gather_body(xf_hbm, idx_hbm, scale_hbm, out_hbm,
                 idx_v, scale_v, gidx_v, buf, sem):
    wid = lax.axis_index("s") * SC_NC + lax.axis_index("c")
    base = wid * RPW
    boff = (base // KTOP) * N
    pltpu.sync_copy(idx_hbm.at[wid], idx_v)
    pltpu.sync_copy(scale_hbm.at[wid], scale_v)
    for c in range(NCH):
        gidx_v[...] = idx_v[c, :] + boff
        pltpu.async_copy(xf_hbm.at[gidx_v], buf, sem).wait()
        for r in range(CH):
            sv = scale_v[c * CH + r, :]

            def colbody(j, _):
                sl = pl.ds(j * SC_L, SC_L)
                buf[r, sl] = buf[r, sl] * sv
                return 0

            lax.fori_loop(0, D // SC_L, colbody, 0, unroll=8)
        pltpu.sync_copy(buf, out_hbm.at[pl.ds(base + c * CH, CH)])


@functools.cache
def _make_gather_scale():
    return functools.partial(
        pl.kernel,
        mesh=plsc.VectorSubcoreMesh(core_axis_name="c", subcore_axis_name="s"),
        out_type=jax.ShapeDtypeStruct((BATCH * KTOP, D), jnp.float32),
        scratch_types=[
            pltpu.VMEM((NCH, SC_L), jnp.int32),
            pltpu.VMEM((RPW, SC_L), jnp.float32),
            pltpu.VMEM((SC_L,), jnp.int32),
            pltpu.VMEM((CH, D), jnp.float32),
            pltpu.SemaphoreType.DMA,
        ],
    )(_gather_body)


def kernel(x, p):
    scol = _scores(x, p.reshape(1, D))
    srow = scol.reshape(BATCH, 1, N)
    idxc, scalec = _topk(scol, srow)
    xf = x.reshape(BATCH * N, D)
    idx3 = idxc.reshape(NW, NCH, SC_L)
    scale3 = scalec.reshape(NW, RPW, SC_L)
    outf = _make_gather_scale()(xf, idx3, scale3)
    return outf.reshape(BATCH, KTOP, D)
